# baseline (speedup 1.0000x reference)
import jax
import jax.numpy as jnp
from jax import lax
from jax.experimental import pallas as pl
from jax.experimental.pallas import tpu as pltpu
from jax.experimental.pallas import tpu_sc as plsc

N = 10000
E = 320000
D = 128

_NC = 2
_NS = 16
_NW = _NC * _NS
_NP = 10112
_ROWS_PER_TILE = _NP // _NS

_CS = 40
_NCH = E // _CS
_STEPS = _NCH // _NW
_TRIPLES = (_STEPS - 4) // 3
_ROW_BYTES = _CS * D * 4
_IDX_BYTES = _CS * 4


def _acc_slices(base0):
    off = 0
    while off < _ROWS_PER_TILE:
        sz = min(_CS, _ROWS_PER_TILE - off)
        yield base0 + off, sz
        off += sz


def _sc_body(nh_hbm, src_hbm, dst_hbm, sums_out, cnt_out,
             rows, is0, is1, is2, ds0, ds1, ds2,
             acc, ls0, ls1, ls2, ss0, ss1, ss2, gsem):
    iss = (is0, is1, is2)
    dss = (ds0, ds1, ds2)
    lss = (ls0, ls1, ls2)
    sss = (ss0, ss1, ss2)
    cid = lax.axis_index("c")
    sid = lax.axis_index("s")
    wid = cid * _NS + sid
    base0 = sid * _ROWS_PER_TILE

    def _fill_slot0(val):
        def _f(r, carry):
            for cc in range(D // 16):
                rows[0, r, pl.ds(cc * 16, 16)] = val
            return carry
        lax.fori_loop(0, _CS, _f, 0)

    def _chunk(i):
        return (wid + i * _NW) * _CS

    def _drain_s(x):
        pltpu.make_async_copy(nh_hbm.at[pl.ds(0, _CS)], rows.at[x], sss[x]).wait()

    def _zero_acc():
        for row, sz in _acc_slices(base0):
            pltpu.sync_copy(rows.at[0, pl.ds(0, sz)], acc.at[pl.ds(row, sz)])

    def _writeout(out_ref):
        for row, sz in _acc_slices(base0):
            pltpu.sync_copy(acc.at[pl.ds(row, sz)], rows.at[1, pl.ds(0, sz)])
            pltpu.sync_copy(rows.at[1, pl.ds(0, sz)],
                            out_ref.at[cid, pl.ds(row, sz)])

    _fill_slot0(jnp.zeros((16,), jnp.float32))
    _zero_acc()
    _fill_slot0(jnp.ones((16,), jnp.float32))
    plsc.subcore_barrier()

    def _c_fire_load(i, u):
        @pl.when(i < _STEPS)
        def _():
            pltpu.async_copy(dst_hbm.at[pl.ds(_chunk(i), _CS)], dss[u], lss[u])

    def _c_step(i, t, drain):
        u = (t + 1) % 3
        if drain:
            _drain_s(u)
        _c_fire_load(i + 1, u)
        pltpu.make_async_copy(dst_hbm.at[pl.ds(0, _CS)], dss[t], lss[t]).wait()
        pltpu.async_copy(rows.at[0], acc.at[dss[t]], sss[t], add=True)

    _c_fire_load(0, 0)
    _c_step(0, 0, False)
    _c_step(1, 1, False)

    def _c_body(k, carry):
        i = 3 * k + 2
        _c_step(i, 2, True)
        _c_step(i + 1, 0, True)
        _c_step(i + 2, 1, True)
        return carry
    lax.fori_loop(0, _TRIPLES, _c_body, 0)
    _c_step(_STEPS - 2, 2, True)
    _c_step(_STEPS - 1, 0, True)
    _drain_s(2)
    _drain_s(0)

    plsc.subcore_barrier()
    _writeout(cnt_out)
    plsc.subcore_barrier()

    _fill_slot0(jnp.zeros((16,), jnp.float32))
    _zero_acc()
    plsc.subcore_barrier()

    def _s_fire_load(i, u):
        @pl.when(i < _STEPS)
        def _():
            base = _chunk(i)
            pltpu.async_copy(src_hbm.at[pl.ds(base, _CS)], iss[u], lss[u])
            pltpu.async_copy(dst_hbm.at[pl.ds(base, _CS)], dss[u], lss[u])

    def _s_step(i, t, drain):
        u = (t + 1) % 3
        if drain:
            _drain_s(u)
        _s_fire_load(i + 1, u)
        pltpu.make_async_copy(src_hbm.at[pl.ds(0, _CS)], iss[t], lss[t]).wait()
        pltpu.make_async_copy(dst_hbm.at[pl.ds(0, _CS)], dss[t], lss[t]).wait()
        gh = pltpu.async_copy(nh_hbm.at[iss[t]], rows.at[t], gsem)
        gh.wait()
        pltpu.async_copy(rows.at[t], acc.at[dss[t]], sss[t], add=True)

    _s_fire_load(0, 0)
    _s_step(0, 0, False)
    _s_step(1, 1, False)

    def _s_body(k, carry):
        i = 3 * k + 2
        _s_step(i, 2, True)
        _s_step(i + 1, 0, True)
        _s_step(i + 2, 1, True)
        return carry
    lax.fori_loop(0, _TRIPLES, _s_body, 0)
    _s_step(_STEPS - 2, 2, True)
    _s_step(_STEPS - 1, 0, True)
    _drain_s(2)
    _drain_s(0)

    plsc.subcore_barrier()
    _writeout(sums_out)


@jax.jit
def _sc_scatter(nh, src, dst):
    mesh = plsc.VectorSubcoreMesh(core_axis_name="c", subcore_axis_name="s")
    return pl.kernel(
        _sc_body,
        out_type=(
            jax.ShapeDtypeStruct((_NC, _NP, D), jnp.float32),
            jax.ShapeDtypeStruct((_NC, _NP, D), jnp.float32),
        ),
        mesh=mesh,
        scratch_types=(
            [pltpu.VMEM((3, _CS, D), jnp.float32)]
            + [pltpu.VMEM((_CS,), jnp.int32) for _ in range(6)]
            + [pltpu.VMEM_SHARED((_NP, D), jnp.float32)]
            + [pltpu.SemaphoreType.DMA for _ in range(7)]
        ),
    )(nh, src, dst)


def _mlp_body(sums_ref, cnt_ref, nh_ref, w1a_ref, w1b_ref, b1_ref,
              w2_ref, b2_ref, out_ref):
    s = sums_ref[0] + sums_ref[1]
    c = cnt_ref[0] + cnt_ref[1]
    cnt = jnp.maximum(c[:, 0:1], 1.0)
    agg = s / cnt
    x = jnp.dot(nh_ref[...], w1a_ref[...], preferred_element_type=jnp.float32)
    x = x + jnp.dot(agg, w1b_ref[...], preferred_element_type=jnp.float32)
    h = jnp.maximum(x + b1_ref[...], 0.0)
    out_ref[...] = (jnp.dot(h, w2_ref[...], preferred_element_type=jnp.float32)
                    + b2_ref[...])


_BLK = 1000


@jax.jit
def _mlp(sums, cnts, nh, w1a, w1b, b1, w2, b2):
    grid = (N // _BLK,)
    return pl.pallas_call(
        _mlp_body,
        grid=grid,
        in_specs=[
            pl.BlockSpec((_NC, _BLK, D), lambda i: (0, i, 0)),
            pl.BlockSpec((_NC, _BLK, D), lambda i: (0, i, 0)),
            pl.BlockSpec((_BLK, D), lambda i: (i, 0)),
            pl.BlockSpec((D, D), lambda i: (0, 0)),
            pl.BlockSpec((D, D), lambda i: (0, 0)),
            pl.BlockSpec((1, D), lambda i: (0, 0)),
            pl.BlockSpec((D, D), lambda i: (0, 0)),
            pl.BlockSpec((1, D), lambda i: (0, 0)),
        ],
        out_specs=pl.BlockSpec((_BLK, D), lambda i: (i, 0)),
        out_shape=jax.ShapeDtypeStruct((N, D), jnp.float32),
    )(sums, cnts, nh, w1a, w1b, b1, w2, b2)


def kernel(nh, eh, edge_index, W1, b1, W2, b2):
    src = edge_index[0]
    dst = edge_index[1]
    sums, cnts = _sc_scatter(nh, src, dst)
    n_h = _mlp(sums, cnts, nh, W1[:D], W1[D:], b1.reshape(1, D),
               W2, b2.reshape(1, D))
    return (n_h, eh)


# speedup vs baseline: 1.2856x; 1.2856x over previous
import jax
import jax.numpy as jnp
from jax import lax
from jax.experimental import pallas as pl
from jax.experimental.pallas import tpu as pltpu
from jax.experimental.pallas import tpu_sc as plsc

N = 10000
E = 320000
D = 128

_NC = 2
_NS = 16
_NW = _NC * _NS
_NP = 10112
_ROWS_PER_TILE = _NP // _NS

_CS = 64
_NCH_S = E // _CS
_ITERS_S = _NCH_S // _NW
_BODIES_S = _ITERS_S // 2
_EXTRA_S = _NCH_S - _ITERS_S * _NW

_CC = 128
_NCH_C = E // _CC
_ITERS_C = _NCH_C // _NW
_BODIES_C = _ITERS_C // 2
_EXTRA_C = _NCH_C - _ITERS_C * _NW


def _acc_slices(base0, piece):
    off = 0
    while off < _ROWS_PER_TILE:
        sz = min(piece, _ROWS_PER_TILE - off)
        yield base0 + off, sz
        off += sz


def _sc_body(nh_hbm, src_hbm, dst_hbm, sums_out, cnt_out,
             ia, ib, da, db, dc, dd, de, df, rows_a, rows_b, acc,
             lsem, ssem, ga, gb):
    cid = lax.axis_index("c")
    sid = lax.axis_index("s")
    wid = cid * _NS + sid
    base0 = sid * _ROWS_PER_TILE

    def _fill(val):
        def _f(r, carry):
            for cc in range(D // 16):
                rows_a[r, pl.ds(cc * 16, 16)] = val
                rows_b[r, pl.ds(cc * 16, 16)] = val
            return carry
        lax.fori_loop(0, _CS, _f, 0)

    _fill(jnp.zeros((16,), jnp.float32))
    for row, sz in _acc_slices(base0, _CS):
        pltpu.sync_copy(rows_a.at[pl.ds(0, sz)], acc.at[pl.ds(row, sz)])
    _fill(jnp.ones((16,), jnp.float32))
    plsc.subcore_barrier()

    def _drain(n):
        for _ in range(n):
            pltpu.make_async_copy(nh_hbm.at[pl.ds(0, _CS)], rows_a, ssem).wait()

    def _cnt_fire(j, q0, q1, q2, q3, drain_first):
        c0 = (wid + (2 * j) * _NW) * _CC
        c1 = (wid + (2 * j + 1) * _NW) * _CC
        hs = (pltpu.async_copy(dst_hbm.at[pl.ds(c0, _CS)], q0, lsem),
              pltpu.async_copy(dst_hbm.at[pl.ds(c0 + _CS, _CS)], q1, lsem),
              pltpu.async_copy(dst_hbm.at[pl.ds(c1, _CS)], q2, lsem),
              pltpu.async_copy(dst_hbm.at[pl.ds(c1 + _CS, _CS)], q3, lsem))
        for h in hs:
            h.wait()
        if drain_first:
            _drain(4)
        pltpu.async_copy(rows_a, acc.at[q0], ssem, add=True)
        pltpu.async_copy(rows_b, acc.at[q1], ssem, add=True)
        pltpu.async_copy(rows_a, acc.at[q2], ssem, add=True)
        pltpu.async_copy(rows_b, acc.at[q3], ssem, add=True)

    _cnt_fire(0, da, db, dc, dd, False)

    def _cnt_body(k, carry):
        _cnt_fire(2 * k + 1, ia, ib, de, df, True)
        _cnt_fire(2 * k + 2, da, db, dc, dd, True)
        return carry
    lax.fori_loop(0, (_BODIES_C - 1) // 2, _cnt_body, 0)
    _drain(4)

    @pl.when(wid < _EXTRA_C)
    def _():
        base = (_ITERS_C * _NW + wid) * _CC
        pltpu.sync_copy(dst_hbm.at[pl.ds(base, _CS)], da)
        pltpu.sync_copy(dst_hbm.at[pl.ds(base + _CS, _CS)], db)
        pltpu.sync_copy(rows_a, acc.at[da], add=True)
        pltpu.sync_copy(rows_b, acc.at[db], add=True)

    def _writeout(out_ref):
        pieces = list(_acc_slices(base0, _CS))
        bufs = (rows_a, rows_b)
        h_in, h_out = {}, {}
        for k, (row, sz) in enumerate(pieces):
            if k >= 2:
                h_out[k - 2].wait()
            h_in[k] = pltpu.async_copy(acc.at[pl.ds(row, sz)],
                                       bufs[k % 2].at[pl.ds(0, sz)], lsem)
            if k >= 1:
                prow, psz = pieces[k - 1]
                h_in[k - 1].wait()
                h_out[k - 1] = pltpu.async_copy(
                    bufs[(k - 1) % 2].at[pl.ds(0, psz)],
                    out_ref.at[cid, pl.ds(prow, psz)], ssem)
        last = len(pieces) - 1
        lrow, lsz = pieces[last]
        h_in[last].wait()
        h_out[last] = pltpu.async_copy(bufs[last % 2].at[pl.ds(0, lsz)],
                                       out_ref.at[cid, pl.ds(lrow, lsz)], ssem)
        h_out[last - 1].wait()
        h_out[last].wait()

    plsc.subcore_barrier()
    _writeout(cnt_out)
    plsc.subcore_barrier()

    def _zrows(r, carry):
        for cc in range(D // 16):
            rows_a[r, pl.ds(cc * 16, 16)] = jnp.zeros((16,), jnp.float32)
        return carry
    lax.fori_loop(0, _CS, _zrows, 0)
    for row, sz in _acc_slices(base0, _CS):
        pltpu.sync_copy(rows_a.at[pl.ds(0, sz)], acc.at[pl.ds(row, sz)])
    plsc.subcore_barrier()

    def _sum_half(q, d0, d1, drain_first):
        c0 = (wid + q * _NW) * _CS
        c1 = (wid + (q + 1) * _NW) * _CS
        hs = (pltpu.async_copy(src_hbm.at[pl.ds(c0, _CS)], ia, lsem),
              pltpu.async_copy(dst_hbm.at[pl.ds(c0, _CS)], d0, lsem),
              pltpu.async_copy(src_hbm.at[pl.ds(c1, _CS)], ib, lsem),
              pltpu.async_copy(dst_hbm.at[pl.ds(c1, _CS)], d1, lsem))
        for h in hs:
            h.wait()
        if drain_first:
            _drain(2)
        gh0 = pltpu.async_copy(nh_hbm.at[ia], rows_a, ga)
        gh1 = pltpu.async_copy(nh_hbm.at[ib], rows_b, gb)
        gh0.wait()
        pltpu.async_copy(rows_a, acc.at[d0], ssem, add=True)
        gh1.wait()
        pltpu.async_copy(rows_b, acc.at[d1], ssem, add=True)

    def _sum_super(k, drain_first):
        _sum_half(4 * k, da, db, drain_first)
        _sum_half(4 * k + 2, dc, dd, True)

    _sum_super(0, False)

    def _sum_body(k, carry):
        _sum_super(k, True)
        return carry
    lax.fori_loop(1, _BODIES_S // 2, _sum_body, 0)
    _drain(2)

    @pl.when(wid < _EXTRA_S)
    def _():
        base = (_ITERS_S * _NW + wid) * _CS
        pltpu.sync_copy(src_hbm.at[pl.ds(base, _CS)], ia)
        pltpu.sync_copy(dst_hbm.at[pl.ds(base, _CS)], da)
        pltpu.async_copy(nh_hbm.at[ia], rows_a, ga).wait()
        pltpu.sync_copy(rows_a, acc.at[da], add=True)

    plsc.subcore_barrier()
    _writeout(sums_out)


@jax.jit
def _sc_scatter(nh, src, dst):
    mesh = plsc.VectorSubcoreMesh(core_axis_name="c", subcore_axis_name="s")
    return pl.kernel(
        _sc_body,
        out_type=(
            jax.ShapeDtypeStruct((_NC, _NP, D), jnp.float32),
            jax.ShapeDtypeStruct((_NC, _NP, D), jnp.float32),
        ),
        mesh=mesh,
        scratch_types=[
            pltpu.VMEM((_CS,), jnp.int32),
            pltpu.VMEM((_CS,), jnp.int32),
            pltpu.VMEM((_CS,), jnp.int32),
            pltpu.VMEM((_CS,), jnp.int32),
            pltpu.VMEM((_CS,), jnp.int32),
            pltpu.VMEM((_CS,), jnp.int32),
            pltpu.VMEM((_CS,), jnp.int32),
            pltpu.VMEM((_CS,), jnp.int32),
            pltpu.VMEM((_CS, D), jnp.float32),
            pltpu.VMEM((_CS, D), jnp.float32),
            pltpu.VMEM_SHARED((_NP, D), jnp.float32),
            pltpu.SemaphoreType.DMA,
            pltpu.SemaphoreType.DMA,
            pltpu.SemaphoreType.DMA,
            pltpu.SemaphoreType.DMA,
        ],
    )(nh, src, dst)


def _mlp_body(sums_ref, cnt_ref, nh_ref, w1a_ref, w1b_ref, b1_ref,
              w2_ref, b2_ref, out_ref):
    s = sums_ref[0] + sums_ref[1]
    c = cnt_ref[0] + cnt_ref[1]
    cnt = jnp.maximum(c[:, 0:1], 1.0)
    agg = s / cnt
    x = jnp.dot(nh_ref[...], w1a_ref[...], preferred_element_type=jnp.float32)
    x = x + jnp.dot(agg, w1b_ref[...], preferred_element_type=jnp.float32)
    h = jnp.maximum(x + b1_ref[...], 0.0)
    out_ref[...] = (jnp.dot(h, w2_ref[...], preferred_element_type=jnp.float32)
                    + b2_ref[...])


_BLK = 1000


@jax.jit
def _mlp(sums, cnts, nh, w1a, w1b, b1, w2, b2):
    grid = (N // _BLK,)
    return pl.pallas_call(
        _mlp_body,
        grid=grid,
        in_specs=[
            pl.BlockSpec((_NC, _BLK, D), lambda i: (0, i, 0)),
            pl.BlockSpec((_NC, _BLK, D), lambda i: (0, i, 0)),
            pl.BlockSpec((_BLK, D), lambda i: (i, 0)),
            pl.BlockSpec((D, D), lambda i: (0, 0)),
            pl.BlockSpec((D, D), lambda i: (0, 0)),
            pl.BlockSpec((1, D), lambda i: (0, 0)),
            pl.BlockSpec((D, D), lambda i: (0, 0)),
            pl.BlockSpec((1, D), lambda i: (0, 0)),
        ],
        out_specs=pl.BlockSpec((_BLK, D), lambda i: (i, 0)),
        out_shape=jax.ShapeDtypeStruct((N, D), jnp.float32),
    )(sums, cnts, nh, w1a, w1b, b1, w2, b2)


def kernel(nh, eh, edge_index, W1, b1, W2, b2):
    src = edge_index[0]
    dst = edge_index[1]
    sums, cnts = _sc_scatter(nh, src, dst)
    n_h = _mlp(sums, cnts, nh, W1[:D], W1[D:], b1.reshape(1, D),
               W2, b2.reshape(1, D))
    return (n_h, eh)


# speedup vs baseline: 1.2910x; 1.0042x over previous
import jax
import jax.numpy as jnp
from jax import lax
from jax.experimental import pallas as pl
from jax.experimental.pallas import tpu as pltpu
from jax.experimental.pallas import tpu_sc as plsc

N = 10000
E = 320000
D = 128

_NC = 2
_NS = 16
_NW = _NC * _NS
_NP = 10112
_ROWS_PER_TILE = _NP // _NS

_CS = 64
_NCH_S = E // _CS
_ITERS_S = _NCH_S // _NW
_BODIES_S = _ITERS_S // 2
_EXTRA_S = _NCH_S - _ITERS_S * _NW

_CC = 128
_NCH_C = E // _CC
_ITERS_C = _NCH_C // _NW
_BODIES_C = _ITERS_C // 2
_EXTRA_C = _NCH_C - _ITERS_C * _NW


def _acc_slices(base0, piece):
    off = 0
    while off < _ROWS_PER_TILE:
        sz = min(piece, _ROWS_PER_TILE - off)
        yield base0 + off, sz
        off += sz


def _sc_body(nh_hbm, src_hbm, dst_hbm, sums_out, cnt_out,
             ia, ib, da, db, dc, dd, de, df, rows_a, rows_b, acc,
             lsem, ssem, ssb, ga, gb):
    cid = lax.axis_index("c")
    sid = lax.axis_index("s")
    wid = cid * _NS + sid
    base0 = sid * _ROWS_PER_TILE

    def _fill(val):
        def _f(r, carry):
            for cc in range(D // 16):
                rows_a[r, pl.ds(cc * 16, 16)] = val
                rows_b[r, pl.ds(cc * 16, 16)] = val
            return carry
        lax.fori_loop(0, _CS, _f, 0)

    _fill(jnp.zeros((16,), jnp.float32))
    for row, sz in _acc_slices(base0, _CS):
        pltpu.sync_copy(rows_a.at[pl.ds(0, sz)], acc.at[pl.ds(row, sz)])
    _fill(jnp.ones((16,), jnp.float32))
    plsc.subcore_barrier()

    def _drain(n):
        for _ in range(n):
            pltpu.make_async_copy(nh_hbm.at[pl.ds(0, _CS)], rows_a, ssem).wait()

    def _cnt_fire(j, q0, q1, q2, q3, drain_first):
        c0 = (wid + (2 * j) * _NW) * _CC
        c1 = (wid + (2 * j + 1) * _NW) * _CC
        hs = (pltpu.async_copy(dst_hbm.at[pl.ds(c0, _CS)], q0, lsem),
              pltpu.async_copy(dst_hbm.at[pl.ds(c0 + _CS, _CS)], q1, lsem),
              pltpu.async_copy(dst_hbm.at[pl.ds(c1, _CS)], q2, lsem),
              pltpu.async_copy(dst_hbm.at[pl.ds(c1 + _CS, _CS)], q3, lsem))
        for h in hs:
            h.wait()
        if drain_first:
            _drain(4)
        pltpu.async_copy(rows_a, acc.at[q0], ssem, add=True)
        pltpu.async_copy(rows_b, acc.at[q1], ssem, add=True)
        pltpu.async_copy(rows_a, acc.at[q2], ssem, add=True)
        pltpu.async_copy(rows_b, acc.at[q3], ssem, add=True)

    _cnt_fire(0, da, db, dc, dd, False)

    def _cnt_body(k, carry):
        _cnt_fire(2 * k + 1, ia, ib, de, df, True)
        _cnt_fire(2 * k + 2, da, db, dc, dd, True)
        return carry
    lax.fori_loop(0, (_BODIES_C - 1) // 2, _cnt_body, 0)
    _drain(4)

    @pl.when(wid < _EXTRA_C)
    def _():
        base = (_ITERS_C * _NW + wid) * _CC
        pltpu.sync_copy(dst_hbm.at[pl.ds(base, _CS)], da)
        pltpu.sync_copy(dst_hbm.at[pl.ds(base + _CS, _CS)], db)
        pltpu.sync_copy(rows_a, acc.at[da], add=True)
        pltpu.sync_copy(rows_b, acc.at[db], add=True)

    def _writeout(out_ref):
        pieces = list(_acc_slices(base0, _CS))
        bufs = (rows_a, rows_b)
        h_in, h_out = {}, {}
        for k, (row, sz) in enumerate(pieces):
            if k >= 2:
                h_out[k - 2].wait()
            h_in[k] = pltpu.async_copy(acc.at[pl.ds(row, sz)],
                                       bufs[k % 2].at[pl.ds(0, sz)], lsem)
            if k >= 1:
                prow, psz = pieces[k - 1]
                h_in[k - 1].wait()
                h_out[k - 1] = pltpu.async_copy(
                    bufs[(k - 1) % 2].at[pl.ds(0, psz)],
                    out_ref.at[cid, pl.ds(prow, psz)], ssem)
        last = len(pieces) - 1
        lrow, lsz = pieces[last]
        h_in[last].wait()
        h_out[last] = pltpu.async_copy(bufs[last % 2].at[pl.ds(0, lsz)],
                                       out_ref.at[cid, pl.ds(lrow, lsz)], ssem)
        h_out[last - 1].wait()
        h_out[last].wait()

    plsc.subcore_barrier()
    _writeout(cnt_out)
    plsc.subcore_barrier()

    def _zrows(r, carry):
        for cc in range(D // 16):
            rows_a[r, pl.ds(cc * 16, 16)] = jnp.zeros((16,), jnp.float32)
        return carry
    lax.fori_loop(0, _CS, _zrows, 0)
    for row, sz in _acc_slices(base0, _CS):
        pltpu.sync_copy(rows_a.at[pl.ds(0, sz)], acc.at[pl.ds(row, sz)])
    plsc.subcore_barrier()

    def _sum_half(q, d0, d1, drain_first):
        c0 = (wid + q * _NW) * _CS
        c1 = (wid + (q + 1) * _NW) * _CS
        hs = (pltpu.async_copy(src_hbm.at[pl.ds(c0, _CS)], ia, lsem),
              pltpu.async_copy(dst_hbm.at[pl.ds(c0, _CS)], d0, lsem),
              pltpu.async_copy(src_hbm.at[pl.ds(c1, _CS)], ib, lsem),
              pltpu.async_copy(dst_hbm.at[pl.ds(c1, _CS)], d1, lsem))
        for h in hs:
            h.wait()
        if drain_first:
            _drain(1)
        gh0 = pltpu.async_copy(nh_hbm.at[ia], rows_a, ga)
        if drain_first:
            pltpu.make_async_copy(nh_hbm.at[pl.ds(0, _CS)], rows_b, ssb).wait()
        gh1 = pltpu.async_copy(nh_hbm.at[ib], rows_b, gb)
        gh0.wait()
        pltpu.async_copy(rows_a, acc.at[d0], ssem, add=True)
        gh1.wait()
        pltpu.async_copy(rows_b, acc.at[d1], ssb, add=True)

    def _sum_super(k, drain_first):
        _sum_half(4 * k, da, db, drain_first)
        _sum_half(4 * k + 2, dc, dd, True)

    _sum_super(0, False)

    def _sum_body(k, carry):
        _sum_super(k, True)
        return carry
    lax.fori_loop(1, _BODIES_S // 2, _sum_body, 0)
    _drain(1)
    pltpu.make_async_copy(nh_hbm.at[pl.ds(0, _CS)], rows_b, ssb).wait()

    @pl.when(wid < _EXTRA_S)
    def _():
        base = (_ITERS_S * _NW + wid) * _CS
        pltpu.sync_copy(src_hbm.at[pl.ds(base, _CS)], ia)
        pltpu.sync_copy(dst_hbm.at[pl.ds(base, _CS)], da)
        pltpu.async_copy(nh_hbm.at[ia], rows_a, ga).wait()
        pltpu.sync_copy(rows_a, acc.at[da], add=True)

    plsc.subcore_barrier()
    _writeout(sums_out)


@jax.jit
def _sc_scatter(nh, src, dst):
    mesh = plsc.VectorSubcoreMesh(core_axis_name="c", subcore_axis_name="s")
    return pl.kernel(
        _sc_body,
        out_type=(
            jax.ShapeDtypeStruct((_NC, _NP, D), jnp.float32),
            jax.ShapeDtypeStruct((_NC, _NP, D), jnp.float32),
        ),
        mesh=mesh,
        scratch_types=[
            pltpu.VMEM((_CS,), jnp.int32),
            pltpu.VMEM((_CS,), jnp.int32),
            pltpu.VMEM((_CS,), jnp.int32),
            pltpu.VMEM((_CS,), jnp.int32),
            pltpu.VMEM((_CS,), jnp.int32),
            pltpu.VMEM((_CS,), jnp.int32),
            pltpu.VMEM((_CS,), jnp.int32),
            pltpu.VMEM((_CS,), jnp.int32),
            pltpu.VMEM((_CS, D), jnp.float32),
            pltpu.VMEM((_CS, D), jnp.float32),
            pltpu.VMEM_SHARED((_NP, D), jnp.float32),
            pltpu.SemaphoreType.DMA,
            pltpu.SemaphoreType.DMA,
            pltpu.SemaphoreType.DMA,
            pltpu.SemaphoreType.DMA,
            pltpu.SemaphoreType.DMA,
        ],
    )(nh, src, dst)


def _mlp_body(sums_ref, cnt_ref, nh_ref, w1a_ref, w1b_ref, b1_ref,
              w2_ref, b2_ref, out_ref):
    s = sums_ref[0] + sums_ref[1]
    c = cnt_ref[0] + cnt_ref[1]
    cnt = jnp.maximum(c[:, 0:1], 1.0)
    agg = s / cnt
    x = jnp.dot(nh_ref[...], w1a_ref[...], preferred_element_type=jnp.float32)
    x = x + jnp.dot(agg, w1b_ref[...], preferred_element_type=jnp.float32)
    h = jnp.maximum(x + b1_ref[...], 0.0)
    out_ref[...] = (jnp.dot(h, w2_ref[...], preferred_element_type=jnp.float32)
                    + b2_ref[...])


_BLK = 1000


@jax.jit
def _mlp(sums, cnts, nh, w1a, w1b, b1, w2, b2):
    grid = (N // _BLK,)
    return pl.pallas_call(
        _mlp_body,
        grid=grid,
        in_specs=[
            pl.BlockSpec((_NC, _BLK, D), lambda i: (0, i, 0)),
            pl.BlockSpec((_NC, _BLK, D), lambda i: (0, i, 0)),
            pl.BlockSpec((_BLK, D), lambda i: (i, 0)),
            pl.BlockSpec((D, D), lambda i: (0, 0)),
            pl.BlockSpec((D, D), lambda i: (0, 0)),
            pl.BlockSpec((1, D), lambda i: (0, 0)),
            pl.BlockSpec((D, D), lambda i: (0, 0)),
            pl.BlockSpec((1, D), lambda i: (0, 0)),
        ],
        out_specs=pl.BlockSpec((_BLK, D), lambda i: (i, 0)),
        out_shape=jax.ShapeDtypeStruct((N, D), jnp.float32),
    )(sums, cnts, nh, w1a, w1b, b1, w2, b2)


def kernel(nh, eh, edge_index, W1, b1, W2, b2):
    src = edge_index[0]
    dst = edge_index[1]
    sums, cnts = _sc_scatter(nh, src, dst)
    n_h = _mlp(sums, cnts, nh, W1[:D], W1[D:], b1.reshape(1, D),
               W2, b2.reshape(1, D))
    return (n_h, eh)
